# Initial kernel scaffold; baseline (speedup 1.0000x reference)
import jax
import jax.numpy as jnp
from jax.experimental import pallas as pl


def kernel(sampling_weights, sparse_i_idx, sparse_j_idx, beta, a, Z, G):
    raise NotImplementedError("write your pallas kernel here")


# speedup vs baseline: 311.5031x; 311.5031x over previous
import functools

import jax
import jax.numpy as jnp
from jax import lax
from jax.experimental import pallas as pl
from jax.experimental.pallas import tpu as pltpu
from jax.experimental.pallas import tpu_sc as plsc

N = 10000
E = 640000
K = 8
S = 1500
SPAD = 1536
NW = 32
EPW = E // NW
ECH = 2000
RB = 128
NRB = SPAD // RB


def _prep_body(w_ref, negg_ref, z_ref, gt_ref, mask_ref, vt_ref):
    w = w_ref[...]
    negg = negg_ref[...]
    p = w / jnp.sum(w)
    g = negg - jnp.log(p)
    bu = lax.bitcast_convert_type(g, jnp.uint32)
    key = jnp.where(bu >> 31 == 1, ~bu, bu | jnp.uint32(0x80000000))

    def bs_body(_, carry):
        lo, hi = carry
        mid = lo + (hi - lo) // 2
        cnt = jnp.sum((key <= mid).astype(jnp.int32))
        take = cnt >= S
        return jnp.where(take, lo, mid + 1), jnp.where(take, mid, hi)

    _, thr = lax.fori_loop(
        0, 32, bs_body, (jnp.uint32(0), jnp.uint32(0xFFFFFFFF)))
    mask = (key <= thr).astype(jnp.float32)

    z = z_ref[...]
    ze = jnp.exp(z - jnp.max(z, axis=0, keepdims=True))
    zs = ze / jnp.sum(ze, axis=0, keepdims=True)
    gt = gt_ref[...]
    gs = 1.0 / (1.0 + jnp.exp(-gt))
    zgt = zs * gs
    ct = zgt / jnp.sum(zgt, axis=1, keepdims=True)
    zsm = zs * mask
    m2 = lax.dot_general(zsm, ct, (((1,), (1,)), ((), ())),
                         preferred_element_type=jnp.float32)
    vt = lax.dot_general(m2, zs, (((1,), (0,)), ((), ())),
                         preferred_element_type=jnp.float32)
    mask_ref[...] = mask
    vt_ref[...] = vt


_prep_call = pl.pallas_call(
    _prep_body,
    out_shape=[
        jax.ShapeDtypeStruct((1, N), jnp.float32),
        jax.ShapeDtypeStruct((K, N), jnp.float32),
    ],
)


def _sc_sqrt(x):
    b = plsc.bitcast(x, jnp.int32)
    y = plsc.bitcast((b >> 1) + jnp.int32(0x1FBD1DF5), jnp.float32)
    y = 0.5 * (y + x / y)
    y = 0.5 * (y + x / y)
    return y


def _sc_body(vt_hbm, beta_hbm, mask_hbm, ii_hbm, jj_hbm,
             vsamp_hbm, bsamp_hbm, acc_hbm,
             vt_v, beta_v, mask_v, ich_v, jch_v, sid_v, vs_v, bs_v, st_v):
    wid = lax.axis_index("s") * 2 + lax.axis_index("c")
    pltpu.sync_copy(vt_hbm, vt_v)
    pltpu.sync_copy(beta_hbm, beta_v)
    pltpu.sync_copy(mask_hbm, mask_v)

    @pl.when(wid == 0)
    def _():
        def z_body(i, c):
            sid_v[pl.ds(i * 16, 16)] = jnp.zeros((16,), jnp.int32)
            return c

        lax.fori_loop(0, SPAD // 16, z_body, 0)

        def comp_body(i, c):
            mv = mask_v[pl.ds(i * 16, 16)]
            sel = mv > 0.5
            seli = sel.astype(jnp.int32)
            pos = c + plsc.cumsum(seli) - 1
            ids = lax.iota(jnp.int32, 16) + i * 16
            okm = sel & (pos < SPAD)
            plsc.store_scatter(sid_v, [pos], ids, mask=okm)
            return c + jnp.sum(seli)

        cnt = lax.fori_loop(0, N // 16, comp_body, jnp.int32(0))

        def gath_body(j, c):
            off = j * 16
            idxv = sid_v[pl.ds(off, 16)]
            posv = lax.iota(jnp.int32, 16) + off
            valid = posv < cnt
            bk = plsc.load_gather(beta_v, [idxv])
            bs_v[pl.ds(off, 16)] = jnp.where(valid, bk, -1e9)
            for k in range(K):
                kk = jnp.full((16,), k, jnp.int32)
                vs_v[pl.ds(k * SPAD + off, 16)] = plsc.load_gather(vt_v, [kk, idxv])
            return c

        lax.fori_loop(0, SPAD // 16, gath_body, 0)
        pltpu.sync_copy(vs_v, vsamp_hbm)
        pltpu.sync_copy(bs_v, bsamp_hbm)

    ebase = wid * EPW

    def chunk_body(ci, carry):
        accb, accd = carry
        base = ebase + ci * ECH
        pltpu.sync_copy(ii_hbm.at[pl.ds(base, ECH)], ich_v)
        pltpu.sync_copy(jj_hbm.at[pl.ds(base, ECH)], jch_v)

        def vec_body(vi, carry2):
            ab, ad = carry2
            off = vi * 16
            ii = ich_v[pl.ds(off, 16)]
            jj = jch_v[pl.ds(off, 16)]
            keep = plsc.load_gather(mask_v, [ii]) * plsc.load_gather(mask_v, [jj])
            bsum = plsc.load_gather(beta_v, [ii]) + plsc.load_gather(beta_v, [jj])
            d2 = jnp.zeros((16,), jnp.float32)
            for k in range(K):
                kk = jnp.full((16,), k, jnp.int32)
                d = (plsc.load_gather(vt_v, [kk, ii])
                     - plsc.load_gather(vt_v, [kk, jj]) + 1e-6)
                d2 = d2 + d * d
            return ab + keep * bsum, ad + keep * _sc_sqrt(d2)

        return lax.fori_loop(0, ECH // 16, vec_body, (accb, accd))

    accb, accd = lax.fori_loop(
        0, EPW // ECH, chunk_body,
        (jnp.zeros((16,), jnp.float32), jnp.zeros((16,), jnp.float32)))
    st_v[pl.ds(0, 16)] = accb
    st_v[pl.ds(16, 16)] = accd
    pltpu.sync_copy(st_v, acc_hbm.at[pl.ds(wid * 32, 32)])


_sc_call = pl.kernel(
    _sc_body,
    out_type=[
        jax.ShapeDtypeStruct((K * SPAD,), jnp.float32),
        jax.ShapeDtypeStruct((SPAD,), jnp.float32),
        jax.ShapeDtypeStruct((NW * 32,), jnp.float32),
    ],
    mesh=plsc.VectorSubcoreMesh(core_axis_name="c", subcore_axis_name="s"),
    scratch_types=[
        pltpu.VMEM((K, N), jnp.float32),
        pltpu.VMEM((N,), jnp.float32),
        pltpu.VMEM((N,), jnp.float32),
        pltpu.VMEM((ECH,), jnp.int32),
        pltpu.VMEM((ECH,), jnp.int32),
        pltpu.VMEM((SPAD,), jnp.int32),
        pltpu.VMEM((K * SPAD,), jnp.float32),
        pltpu.VMEM((SPAD,), jnp.float32),
        pltpu.VMEM((32,), jnp.float32),
    ],
    compiler_params=pltpu.CompilerParams(needs_layout_passes=False),
)


def _dense_body(vs_ref, bs_ref, acc_ref, a_ref, out_ref):
    i = pl.program_id(0)
    at_full = vs_ref[...]
    bs = bs_ref[...]
    atr = vs_ref[:, pl.ds(i * RB, RB)]
    bsr = bs_ref[:, pl.ds(i * RB, RB)]

    ones_t = jnp.ones((1, SPAD), jnp.float32)
    cdims = (((0,), (0,)), ((), ()))
    nt = jnp.sum(at_full * at_full, axis=0, keepdims=True)
    rt = jnp.sum(at_full, axis=0, keepdims=True)
    nr = lax.dot_general(jnp.sum(atr * atr, axis=0, keepdims=True), ones_t,
                         cdims, preferred_element_type=jnp.float32)
    rr = lax.dot_general(jnp.sum(atr, axis=0, keepdims=True), ones_t,
                         cdims, preferred_element_type=jnp.float32)
    br = lax.dot_general(bsr, ones_t, cdims,
                         preferred_element_type=jnp.float32)
    p = lax.dot_general(atr, at_full, cdims,
                        preferred_element_type=jnp.float32)

    a = a_ref[0]
    sa = jnp.maximum(a, 0.0) + jnp.log(1.0 + jnp.exp(-jnp.abs(a)))

    d2 = nr + nt - 2.0 * p + 2e-6 * (rr - rt) + 8e-12
    dist = jnp.sqrt(jnp.maximum(d2, 0.0))
    mat = jnp.exp(br + bs - sa * dist)
    rowi = lax.broadcasted_iota(jnp.int32, (RB, SPAD), 0) + i * RB
    coli = lax.broadcasted_iota(jnp.int32, (RB, SPAD), 1)
    s_off = jnp.sum(jnp.where(rowi == coli, 0.0, mat))
    e1 = jnp.exp(jnp.float32(1.0))
    part = 0.5 * (e1 * e1) * s_off

    @pl.when(i == 0)
    def _():
        sb = jnp.sum(acc_ref[:, 0:16])
        sd = jnp.sum(acc_ref[:, 16:32])
        out_ref[0, 0] = (sb - sa * sd) - part

    @pl.when(i > 0)
    def _():
        out_ref[0, 0] = out_ref[0, 0] - part


_dense_call = pl.pallas_call(
    _dense_body,
    grid=(NRB,),
    in_specs=[
        pl.BlockSpec((K, SPAD), lambda i: (0, 0)),
        pl.BlockSpec((1, SPAD), lambda i: (0, 0)),
        pl.BlockSpec((NW, 32), lambda i: (0, 0)),
        pl.BlockSpec(memory_space=pltpu.SMEM),
    ],
    out_specs=pl.BlockSpec(memory_space=pltpu.SMEM),
    out_shape=jax.ShapeDtypeStruct((1, 1), jnp.float32),
)


def kernel(sampling_weights, sparse_i_idx, sparse_j_idx, beta, a, Z, G):
    negg = -jax.random.gumbel(jax.random.key(123), (N,), jnp.float32)
    maskf, vt = _prep_call(
        sampling_weights.reshape(1, N), negg.reshape(1, N), Z, G.T)
    vsamp, bsamp, acc = _sc_call(
        vt, beta, maskf.reshape(N), sparse_i_idx, sparse_j_idx)
    out = _dense_call(vsamp.reshape(K, SPAD), bsamp.reshape(1, SPAD),
                      acc.reshape(NW, 32), a)
    return out[0, 0]
